# baseline (speedup 1.0000x reference)
import functools

import jax
import jax.numpy as jnp
from jax import lax
from jax.experimental import pallas as pl
from jax.experimental.pallas import tpu as pltpu
from jax.experimental.pallas import tpu_sc as plsc

_K = 32
_BN = 1000
_CH = 80
_NCH = 250
_NS = 16
_RPS = 640
_SCN = 10240
_ACCR = 5120
_PASS = 5104
_BCH = 50


def _maxk(h):
    u = lax.bitcast_convert_type(h, jnp.uint32)
    neg = u >= jnp.uint32(0x80000000)
    key = jnp.where(neg, ~u, u | jnp.uint32(0x80000000))
    thresh = jnp.zeros((h.shape[0], 1), jnp.uint32)
    for bit in range(31, -1, -1):
        cand = thresh | jnp.uint32(1 << bit)
        cnt = jnp.sum((key >= cand).astype(jnp.int32), axis=1, keepdims=True)
        thresh = jnp.where(cnt >= _K, cand, thresh)
    return jnp.where(key >= thresh, h, jnp.zeros_like(h))


def _in_body(x_ref, wt_ref, b_ref, o0_ref, o1_ref):
    h = jnp.dot(x_ref[...], wt_ref[...], preferred_element_type=jnp.float32)
    h = _maxk(h + b_ref[...])
    o0_ref[...] = h[:, :128]
    o1_ref[...] = h[:, 128:]


def _tc_in(x, wt, b):
    n = x.shape[0]
    return pl.pallas_call(
        _in_body,
        grid=(n // _BN,),
        in_specs=[
            pl.BlockSpec((_BN, x.shape[1]), lambda i: (i, 0)),
            pl.BlockSpec(wt.shape, lambda i: (0, 0)),
            pl.BlockSpec(b.shape, lambda i: (0, 0)),
        ],
        out_specs=[pl.BlockSpec((_BN, 128), lambda i: (i, 0))] * 2,
        out_shape=[jax.ShapeDtypeStruct((n, 128), jnp.float32)] * 2,
    )(x, wt, b)


def _update(h0, h1, s0, s1, p0, p1, wst, wnt, b):
    cnt = p0[:, :1] + p1[:, :1]
    inv = 1.0 / jnp.maximum(cnt, 1.0)
    a0 = s0 * inv
    a1 = s1 * inv
    f32 = jnp.float32
    h = jnp.dot(h0, wst[:128], preferred_element_type=f32)
    h += jnp.dot(h1, wst[128:], preferred_element_type=f32)
    h += jnp.dot(a0, wnt[:128], preferred_element_type=f32)
    h += jnp.dot(a1, wnt[128:], preferred_element_type=f32)
    return h + b


def _mid_body(h0_ref, h1_ref, s0_ref, s1_ref, p0_ref, p1_ref, wst_ref, wnt_ref,
              b_ref, o0_ref, o1_ref):
    h = _update(h0_ref[...], h1_ref[...], s0_ref[...], s1_ref[...], p0_ref[...],
                p1_ref[...], wst_ref[...], wnt_ref[...], b_ref[...])
    h = _maxk(h)
    o0_ref[...] = h[:, :128]
    o1_ref[...] = h[:, 128:]


def _tc_mid(h0, h1, s0, s1, p0, p1, wst, wnt, b):
    n = h0.shape[0]
    row = lambda i: (i, 0)
    fix = lambda i: (0, 0)
    return pl.pallas_call(
        _mid_body,
        grid=(n // _BN,),
        in_specs=[
            pl.BlockSpec((_BN, 128), row), pl.BlockSpec((_BN, 128), row),
            pl.BlockSpec((_BN, 128), row), pl.BlockSpec((_BN, 128), row),
            pl.BlockSpec((_BN, 128), row), pl.BlockSpec((_BN, 128), row),
            pl.BlockSpec(wst.shape, fix), pl.BlockSpec(wnt.shape, fix),
            pl.BlockSpec(b.shape, fix),
        ],
        out_specs=[pl.BlockSpec((_BN, 128), row)] * 2,
        out_shape=[jax.ShapeDtypeStruct((n, 128), jnp.float32)] * 2,
    )(h0, h1, s0, s1, p0, p1, wst, wnt, b)


def _last_body(h0_ref, h1_ref, s0_ref, s1_ref, p0_ref, p1_ref, wst_ref, wnt_ref,
               b_ref, wot_ref, bo_ref, o_ref):
    h = _update(h0_ref[...], h1_ref[...], s0_ref[...], s1_ref[...], p0_ref[...],
                p1_ref[...], wst_ref[...], wnt_ref[...], b_ref[...])
    o_ref[...] = jnp.dot(h, wot_ref[...], preferred_element_type=jnp.float32) + bo_ref[...]


def _tc_last(h0, h1, s0, s1, p0, p1, wst, wnt, b, wot, bo):
    n = h0.shape[0]
    row = lambda i: (i, 0)
    fix = lambda i: (0, 0)
    return pl.pallas_call(
        _last_body,
        grid=(n // _BN,),
        in_specs=[
            pl.BlockSpec((_BN, 128), row), pl.BlockSpec((_BN, 128), row),
            pl.BlockSpec((_BN, 128), row), pl.BlockSpec((_BN, 128), row),
            pl.BlockSpec((_BN, 128), row), pl.BlockSpec((_BN, 128), row),
            pl.BlockSpec(wst.shape, fix), pl.BlockSpec(wnt.shape, fix),
            pl.BlockSpec(b.shape, fix), pl.BlockSpec(wot.shape, fix),
            pl.BlockSpec(bo.shape, fix),
        ],
        out_specs=pl.BlockSpec((_BN, wot.shape[1]), row),
        out_shape=jax.ShapeDtypeStruct((n, wot.shape[1]), jnp.float32),
    )(h0, h1, s0, s1, p0, p1, wst, wnt, b, wot, bo)


def _sc_bin(src2, dst2):
    mesh = plsc.VectorSubcoreMesh(core_axis_name="c", subcore_axis_name="s")
    i32 = jnp.int32

    @functools.partial(
        pl.kernel,
        out_type=tuple(jax.ShapeDtypeStruct((32, 10240), i32) for _ in range(4))
        + (jax.ShapeDtypeStruct((32, 128), i32),),
        mesh=mesh,
        compiler_params=pltpu.CompilerParams(needs_layout_passes=False),
        scratch_types=[
            pltpu.VMEM((10000,), i32),
            pltpu.VMEM((10000,), i32),
            pltpu.VMEM((10256,), i32),
            pltpu.VMEM((10256,), i32),
            pltpu.VMEM((10256,), i32),
            pltpu.VMEM((10256,), i32),
            pltpu.VMEM((128,), i32),
        ],
    )
    def bink(src_hbm, dst_hbm, sa_hbm, da_hbm, sb_hbm, db_hbm, cn_hbm,
             in_s, in_d, o_sa, o_da, o_sb, o_db, cntv):
        c = lax.axis_index("c")
        s = lax.axis_index("s")
        w = c * _NS + s
        pltpu.sync_copy(src_hbm.at[w], in_s)
        pltpu.sync_copy(dst_hbm.at[w], in_d)

        def step(i, carry):
            ca, cb = carry
            sv = in_s[pl.ds(i * 16, 16)]
            dv = in_d[pl.ds(i * 16, 16)]
            ma = dv < _PASS
            mai = ma.astype(jnp.int32)
            inca = plsc.cumsum(mai)
            incb = plsc.cumsum(jnp.int32(1) - mai)
            posa = ca + (inca - mai)
            posb = cb + (incb - (jnp.int32(1) - mai))
            dump = jnp.full((16,), 10255, jnp.int32)
            ia = jnp.where(ma, posa, dump)
            ib = jnp.where(ma, dump, posb)
            plsc.store_scatter(o_sa, [ia], sv)
            plsc.store_scatter(o_da, [ia], dv)
            plsc.store_scatter(o_sb, [ib], sv)
            plsc.store_scatter(o_db, [ib], dv - _PASS)
            pc = jnp.max(inca)
            return ca + pc, cb + (jnp.int32(16) - pc)
        ca, cb = lax.fori_loop(0, 625, step, (jnp.int32(0), jnp.int32(0)))

        trash = jnp.int32(_PASS) + lax.iota(i32, 16)
        zsrc = jnp.zeros((16,), i32)

        def pad(buf_s, buf_d, cur):
            tgt = ((cur + 239) // 240) * 240
            nst = (tgt - cur + 15) // 16

            def pb(t, _):
                buf_s[pl.ds(cur + t * 16, 16)] = zsrc
                buf_d[pl.ds(cur + t * 16, 16)] = trash
                return 0
            lax.fori_loop(0, nst, pb, 0)
            return tgt // 240
        nba = pad(o_sa, o_da, ca)
        nbb = pad(o_sb, o_db, cb)

        cntv[pl.ds(0, 16)] = jnp.full((16,), nba, i32)
        cntv[pl.ds(16, 16)] = jnp.full((16,), nbb, i32)
        for j in range(2, 8):
            cntv[pl.ds(j * 16, 16)] = zsrc
        pltpu.sync_copy(o_sa.at[pl.ds(0, 10240)], sa_hbm.at[w])
        pltpu.sync_copy(o_da.at[pl.ds(0, 10240)], da_hbm.at[w])
        pltpu.sync_copy(o_sb.at[pl.ds(0, 10240)], sb_hbm.at[w])
        pltpu.sync_copy(o_db.at[pl.ds(0, 10240)], db_hbm.at[w])
        pltpu.sync_copy(cntv, cn_hbm.at[w])

    return bink(src2, dst2)


def _sc_agg(h0, h1, sa4, da4, sb4, db4, cn):
    n = h0.shape[0]
    mesh = plsc.VectorSubcoreMesh(core_axis_name="c", subcore_axis_name="s")
    f32 = jnp.float32

    @functools.partial(
        pl.kernel,
        out_type=(jax.ShapeDtypeStruct((n, 128), f32),
                  jax.ShapeDtypeStruct((n, 128), f32)),
        mesh=mesh,
        compiler_params=pltpu.CompilerParams(needs_layout_passes=False),
        scratch_types=[
            pltpu.VMEM((128, _CH), jnp.int32),
            pltpu.VMEM((128, _CH), jnp.int32),
            pltpu.VMEM((2, 128), jnp.int32),
            pltpu.VMEM((_CH, 128), f32),
            pltpu.VMEM((_CH, 128), f32),
            pltpu.VMEM((_CH, 128), f32),
            pltpu.VMEM((16, 128), f32),
            pltpu.VMEM_SHARED((_ACCR, 128), f32),
            pltpu.SemaphoreType.DMA,
            pltpu.SemaphoreType.DMA,
            pltpu.SemaphoreType.DMA,
            pltpu.SemaphoreType.DMA,
            pltpu.SemaphoreType.DMA,
            pltpu.SemaphoreType.DMA,
        ],
    )
    def agg(h0_hbm, h1_hbm, sa_hbm, da_hbm, sb_hbm, db_hbm, cn_hbm,
            o0_hbm, o1_hbm, isv, idv, cv, rows0, rows1, rows2, zeros_v, acc,
            gs0, gs1, gs2, ss0, ss1, ss2):
        c = lax.axis_index("c")
        s = lax.axis_index("s")
        zv = jnp.zeros((16,), f32)
        for i in range(16):
            for j in range(8):
                zeros_v[i, pl.ds(j * 16, 16)] = zv
        pltpu.sync_copy(cn_hbm.at[pl.ds(2 * s, 2)], cv)
        na0 = jnp.max(cv[0, pl.ds(0, 16)])
        na1 = jnp.max(cv[1, pl.ds(0, 16)])
        nb0 = jnp.max(cv[0, pl.ds(16, 16)])
        nb1 = jnp.max(cv[1, pl.ds(16, 16)])
        rbuf = (rows0, rows1, rows2)
        gsem = (gs0, gs1, gs2)
        ssem = (ss0, ss1, ss2)

        def run(h_hbm, o_hbm, s_hbm, d_hbm, p, n0, n1):
            def zero_body(i, _):
                pltpu.sync_copy(zeros_v, acc.at[pl.ds(s * 320 + i * 16, 16)])
                return 0
            lax.fori_loop(0, 20, zero_body, 0)
            plsc.subcore_barrier()

            def start_g(j, k):
                pltpu.make_async_copy(h_hbm.at[isv.at[j]], rbuf[k],
                                      gsem[k]).start()

            def wait_g(j, k):
                pltpu.make_async_copy(h_hbm.at[isv.at[j]], rbuf[k],
                                      gsem[k]).wait()

            def start_s(j, k):
                pltpu.make_async_copy(rbuf[k], acc.at[idv.at[j]],
                                      ssem[k]).start(add=True)

            def wait_s(j, k):
                pltpu.make_async_copy(rbuf[k], acc.at[idv.at[j]],
                                      ssem[k]).wait()

            for seg in range(2):
                w = 2 * s + seg
                ntr = n0 if seg == 0 else n1
                pltpu.sync_copy(s_hbm.at[w], isv)
                pltpu.sync_copy(d_hbm.at[w], idv)

                @pl.when(ntr > 0)
                def _():
                    m = 3 * ntr
                    start_g(0, 0)
                    wait_g(0, 0)
                    start_s(0, 0)
                    start_g(1, 1)
                    wait_g(1, 1)
                    start_s(1, 1)
                    start_g(2, 2)

                    def body(t, _):
                        for k in range(3):
                            j = 3 * t + 2 + k
                            kk = (2 + k) % 3
                            wait_g(j, kk)
                            start_s(j, kk)
                            wait_s(j - 2, (kk + 1) % 3)
                            start_g(j + 1, (kk + 1) % 3)
                        return 0
                    lax.fori_loop(0, ntr - 1, body, 0)
                    wait_g(m - 1, 2)
                    start_s(m - 1, 2)
                    wait_s(m - 3, 0)
                    wait_s(m - 2, 1)
                    wait_s(m - 1, 2)
            plsc.subcore_barrier()
            tail = 304 if p == 0 else 96

            @pl.when(s < _NS - 1)
            def _():
                pltpu.sync_copy(acc.at[pl.ds(s * 320, 320)],
                                o_hbm.at[pl.ds(p * _PASS + s * 320, 320)])

            @pl.when(s == _NS - 1)
            def _():
                pltpu.sync_copy(acc.at[pl.ds(4800, tail)],
                                o_hbm.at[pl.ds(p * _PASS + 4800, tail)])
            plsc.subcore_barrier()

        pl.when(c == 0)(lambda: run(h0_hbm, o0_hbm, sa_hbm, da_hbm, 0, na0, na1))
        pl.when(c == 1)(lambda: run(h1_hbm, o1_hbm, sa_hbm, da_hbm, 0, na0, na1))
        pl.when(c == 0)(lambda: run(h0_hbm, o0_hbm, sb_hbm, db_hbm, 1, nb0, nb1))
        pl.when(c == 1)(lambda: run(h1_hbm, o1_hbm, sb_hbm, db_hbm, 1, nb0, nb1))

    return agg(h0, h1, sa4, da4, sb4, db4, cn)


def _sc_degree(da4, db4, cn, n):
    mesh = plsc.VectorSubcoreMesh(core_axis_name="c", subcore_axis_name="s")
    f32 = jnp.float32

    @functools.partial(
        pl.kernel,
        out_type=(jax.ShapeDtypeStruct((n, 128), f32),
                  jax.ShapeDtypeStruct((n, 128), f32)),
        mesh=mesh,
        compiler_params=pltpu.CompilerParams(needs_layout_passes=False),
        scratch_types=[
            pltpu.VMEM((128, _CH), jnp.int32),
            pltpu.VMEM((128,), jnp.int32),
            pltpu.VMEM((_CH, 128), f32),
            pltpu.VMEM((16, 128), f32),
            pltpu.VMEM_SHARED((_ACCR, 128), f32),
        ],
    )
    def deg(da_hbm, db_hbm, cn_hbm, p0_hbm, p1_hbm, idv, cv, ones_v, zeros_v,
            acc):
        c = lax.axis_index("c")
        s = lax.axis_index("s")
        w = c * _NS + s
        one = jnp.ones((16,), f32)
        zv = jnp.zeros((16,), f32)
        for i in range(_CH):
            for j in range(8):
                ones_v[i, pl.ds(j * 16, 16)] = one
        for i in range(16):
            for j in range(8):
                zeros_v[i, pl.ds(j * 16, 16)] = zv
        pltpu.sync_copy(cn_hbm.at[w], cv)
        na = jnp.max(cv[pl.ds(0, 16)])
        nb = jnp.max(cv[pl.ds(16, 16)])

        def run(o_hbm, d_hbm, nbatch, p):
            def zero_body(i, _):
                pltpu.sync_copy(zeros_v, acc.at[pl.ds(s * 320 + i * 16, 16)])
                return 0
            lax.fori_loop(0, 20, zero_body, 0)
            plsc.subcore_barrier()

            pltpu.sync_copy(d_hbm.at[w], idv)

            def chunk_body(j, _):
                pltpu.sync_copy(ones_v, acc.at[idv.at[j]], add=True)
                return 0
            lax.fori_loop(0, 3 * nbatch, chunk_body, 0)
            plsc.subcore_barrier()
            tail = 304 if p == 0 else 96

            @pl.when(s < _NS - 1)
            def _():
                pltpu.sync_copy(acc.at[pl.ds(s * 320, 320)],
                                o_hbm.at[pl.ds(p * _PASS + s * 320, 320)])

            @pl.when(s == _NS - 1)
            def _():
                pltpu.sync_copy(acc.at[pl.ds(4800, tail)],
                                o_hbm.at[pl.ds(p * _PASS + 4800, tail)])
            plsc.subcore_barrier()

        pl.when(c == 0)(lambda: run(p0_hbm, da_hbm, na, 0))
        pl.when(c == 1)(lambda: run(p1_hbm, da_hbm, na, 0))
        pl.when(c == 0)(lambda: run(p0_hbm, db_hbm, nb, 1))
        pl.when(c == 1)(lambda: run(p1_hbm, db_hbm, nb, 1))

    return deg(da4, db4, cn)


def kernel(x, edge_index, W_in, b_in, W_neigh0, W_self0, b_self0,
           W_neigh1, W_self1, b_self1, W_out, b_out):
    n = x.shape[0]
    e = edge_index.shape[1]
    assert e == _NS * _NCH * _CH and n % _BN == 0 and n <= _ACCR * 2
    src = edge_index[0]
    dst = edge_index[1]
    src2 = src.reshape(2 * _NS, e // (2 * _NS))
    dst2 = dst.reshape(2 * _NS, e // (2 * _NS))

    sa, da, sb, db, cn = _sc_bin(src2, dst2)
    sa4 = sa.reshape(2 * _NS, 128, _CH)
    da4 = da.reshape(2 * _NS, 128, _CH)
    sb4 = sb.reshape(2 * _NS, 128, _CH)
    db4 = db.reshape(2 * _NS, 128, _CH)

    p0, p1 = _sc_degree(da4, db4, cn, n)
    h0, h1 = _tc_in(x, W_in.T, b_in.reshape(1, -1))
    s0, s1 = _sc_agg(h0, h1, sa4, da4, sb4, db4, cn)
    h0, h1 = _tc_mid(h0, h1, s0, s1, p0, p1, W_self0.T, W_neigh0.T,
                     b_self0.reshape(1, -1))
    s0, s1 = _sc_agg(h0, h1, sa4, da4, sb4, db4, cn)
    return _tc_last(h0, h1, s0, s1, p0, p1, W_self1.T, W_neigh1.T,
                    b_self1.reshape(1, -1), W_out.T, b_out.reshape(1, -1))


# speedup vs baseline: 1.2443x; 1.2443x over previous
import functools

import jax
import jax.numpy as jnp
from jax import lax
from jax.experimental import pallas as pl
from jax.experimental.pallas import tpu as pltpu
from jax.experimental.pallas import tpu_sc as plsc

_K = 32
_BN = 1000
_CH = 80
_NCH = 250
_NS = 16
_RPS = 640
_SCN = 10240
_ACCR = 5120
_PASS = 5104
_BCH = 50


def _maxk(h):
    u = lax.bitcast_convert_type(h, jnp.uint32)
    neg = u >= jnp.uint32(0x80000000)
    key = jnp.where(neg, ~u, u | jnp.uint32(0x80000000))
    thresh = jnp.zeros((h.shape[0], 1), jnp.uint32)
    for bit in range(31, -1, -1):
        cand = thresh | jnp.uint32(1 << bit)
        cnt = jnp.sum((key >= cand).astype(jnp.int32), axis=1, keepdims=True)
        thresh = jnp.where(cnt >= _K, cand, thresh)
    return jnp.where(key >= thresh, h, jnp.zeros_like(h))


def _in_body(x_ref, wt_ref, b_ref, o0_ref, o1_ref):
    h = jnp.dot(x_ref[...], wt_ref[...], preferred_element_type=jnp.float32)
    h = _maxk(h + b_ref[...])
    o0_ref[...] = h[:, :128]
    o1_ref[...] = h[:, 128:]


def _tc_in(x, wt, b):
    n = x.shape[0]
    return pl.pallas_call(
        _in_body,
        grid=(n // _BN,),
        in_specs=[
            pl.BlockSpec((_BN, x.shape[1]), lambda i: (i, 0)),
            pl.BlockSpec(wt.shape, lambda i: (0, 0)),
            pl.BlockSpec(b.shape, lambda i: (0, 0)),
        ],
        out_specs=[pl.BlockSpec((_BN, 128), lambda i: (i, 0))] * 2,
        out_shape=[jax.ShapeDtypeStruct((n, 128), jnp.float32)] * 2,
    )(x, wt, b)


def _update(h0, h1, s0, s1, p0, p1, wst, wnt, b):
    cnt = p0[:, :1] + p1[:, :1]
    inv = 1.0 / jnp.maximum(cnt, 1.0)
    a0 = s0 * inv
    a1 = s1 * inv
    f32 = jnp.float32
    h = jnp.dot(h0, wst[:128], preferred_element_type=f32)
    h += jnp.dot(h1, wst[128:], preferred_element_type=f32)
    h += jnp.dot(a0, wnt[:128], preferred_element_type=f32)
    h += jnp.dot(a1, wnt[128:], preferred_element_type=f32)
    return h + b


def _mid_body(h0_ref, h1_ref, s0_ref, s1_ref, p0_ref, p1_ref, wst_ref, wnt_ref,
              b_ref, o0_ref, o1_ref):
    h = _update(h0_ref[...], h1_ref[...], s0_ref[...], s1_ref[...], p0_ref[...],
                p1_ref[...], wst_ref[...], wnt_ref[...], b_ref[...])
    h = _maxk(h)
    o0_ref[...] = h[:, :128]
    o1_ref[...] = h[:, 128:]


def _tc_mid(h0, h1, s0, s1, p0, p1, wst, wnt, b):
    n = h0.shape[0]
    row = lambda i: (i, 0)
    fix = lambda i: (0, 0)
    return pl.pallas_call(
        _mid_body,
        grid=(n // _BN,),
        in_specs=[
            pl.BlockSpec((_BN, 128), row), pl.BlockSpec((_BN, 128), row),
            pl.BlockSpec((_BN, 128), row), pl.BlockSpec((_BN, 128), row),
            pl.BlockSpec((_BN, 128), row), pl.BlockSpec((_BN, 128), row),
            pl.BlockSpec(wst.shape, fix), pl.BlockSpec(wnt.shape, fix),
            pl.BlockSpec(b.shape, fix),
        ],
        out_specs=[pl.BlockSpec((_BN, 128), row)] * 2,
        out_shape=[jax.ShapeDtypeStruct((n, 128), jnp.float32)] * 2,
    )(h0, h1, s0, s1, p0, p1, wst, wnt, b)


def _last_body(h0_ref, h1_ref, s0_ref, s1_ref, p0_ref, p1_ref, wst_ref, wnt_ref,
               b_ref, wot_ref, bo_ref, o_ref):
    h = _update(h0_ref[...], h1_ref[...], s0_ref[...], s1_ref[...], p0_ref[...],
                p1_ref[...], wst_ref[...], wnt_ref[...], b_ref[...])
    o_ref[...] = jnp.dot(h, wot_ref[...], preferred_element_type=jnp.float32) + bo_ref[...]


def _tc_last(h0, h1, s0, s1, p0, p1, wst, wnt, b, wot, bo):
    n = h0.shape[0]
    row = lambda i: (i, 0)
    fix = lambda i: (0, 0)
    return pl.pallas_call(
        _last_body,
        grid=(n // _BN,),
        in_specs=[
            pl.BlockSpec((_BN, 128), row), pl.BlockSpec((_BN, 128), row),
            pl.BlockSpec((_BN, 128), row), pl.BlockSpec((_BN, 128), row),
            pl.BlockSpec((_BN, 128), row), pl.BlockSpec((_BN, 128), row),
            pl.BlockSpec(wst.shape, fix), pl.BlockSpec(wnt.shape, fix),
            pl.BlockSpec(b.shape, fix), pl.BlockSpec(wot.shape, fix),
            pl.BlockSpec(bo.shape, fix),
        ],
        out_specs=pl.BlockSpec((_BN, wot.shape[1]), row),
        out_shape=jax.ShapeDtypeStruct((n, wot.shape[1]), jnp.float32),
    )(h0, h1, s0, s1, p0, p1, wst, wnt, b, wot, bo)


def _sc_bin(src2, dst2):
    mesh = plsc.VectorSubcoreMesh(core_axis_name="c", subcore_axis_name="s")
    i32 = jnp.int32

    @functools.partial(
        pl.kernel,
        out_type=tuple(jax.ShapeDtypeStruct((32, 10240), i32) for _ in range(4))
        + (jax.ShapeDtypeStruct((32, 128), i32),),
        mesh=mesh,
        compiler_params=pltpu.CompilerParams(needs_layout_passes=False),
        scratch_types=[
            pltpu.VMEM((10000,), i32),
            pltpu.VMEM((10000,), i32),
            pltpu.VMEM((10256,), i32),
            pltpu.VMEM((10256,), i32),
            pltpu.VMEM((10256,), i32),
            pltpu.VMEM((10256,), i32),
            pltpu.VMEM((128,), i32),
        ],
    )
    def bink(src_hbm, dst_hbm, sa_hbm, da_hbm, sb_hbm, db_hbm, cn_hbm,
             in_s, in_d, o_sa, o_da, o_sb, o_db, cntv):
        c = lax.axis_index("c")
        s = lax.axis_index("s")
        w = c * _NS + s
        pltpu.sync_copy(src_hbm.at[w], in_s)
        pltpu.sync_copy(dst_hbm.at[w], in_d)

        def step(i, carry):
            ca, cb = carry
            sv = in_s[pl.ds(i * 16, 16)]
            dv = in_d[pl.ds(i * 16, 16)]
            ma = dv < _PASS
            mai = ma.astype(jnp.int32)
            inca = plsc.cumsum(mai)
            incb = plsc.cumsum(jnp.int32(1) - mai)
            posa = ca + (inca - mai)
            posb = cb + (incb - (jnp.int32(1) - mai))
            dump = jnp.full((16,), 10255, jnp.int32)
            ia = jnp.where(ma, posa, dump)
            ib = jnp.where(ma, dump, posb)
            plsc.store_scatter(o_sa, [ia], sv)
            plsc.store_scatter(o_da, [ia], dv)
            plsc.store_scatter(o_sb, [ib], sv)
            plsc.store_scatter(o_db, [ib], dv - _PASS)
            pc = jnp.max(inca)
            return ca + pc, cb + (jnp.int32(16) - pc)
        ca, cb = lax.fori_loop(0, 625, step, (jnp.int32(0), jnp.int32(0)))

        trash = jnp.int32(_PASS) + lax.iota(i32, 16)
        zsrc = jnp.zeros((16,), i32)

        def pad(buf_s, buf_d, cur):
            tgt = ((cur + 159) // 160) * 160
            nst = (tgt - cur + 15) // 16

            def pb(t, _):
                buf_s[pl.ds(cur + t * 16, 16)] = zsrc
                buf_d[pl.ds(cur + t * 16, 16)] = trash
                return 0
            lax.fori_loop(0, nst, pb, 0)
            return tgt // 160
        nba = pad(o_sa, o_da, ca)
        nbb = pad(o_sb, o_db, cb)

        cntv[pl.ds(0, 16)] = jnp.full((16,), nba, i32)
        cntv[pl.ds(16, 16)] = jnp.full((16,), nbb, i32)
        for j in range(2, 8):
            cntv[pl.ds(j * 16, 16)] = zsrc
        pltpu.sync_copy(o_sa.at[pl.ds(0, 10240)], sa_hbm.at[w])
        pltpu.sync_copy(o_da.at[pl.ds(0, 10240)], da_hbm.at[w])
        pltpu.sync_copy(o_sb.at[pl.ds(0, 10240)], sb_hbm.at[w])
        pltpu.sync_copy(o_db.at[pl.ds(0, 10240)], db_hbm.at[w])
        pltpu.sync_copy(cntv, cn_hbm.at[w])

    return bink(src2, dst2)


def _sc_agg(h0, h1, sa4, da4, sb4, db4, cn):
    n = h0.shape[0]
    mesh = plsc.VectorSubcoreMesh(core_axis_name="c", subcore_axis_name="s")
    f32 = jnp.float32

    @functools.partial(
        pl.kernel,
        out_type=(jax.ShapeDtypeStruct((n, 128), f32),
                  jax.ShapeDtypeStruct((n, 128), f32)),
        mesh=mesh,
        compiler_params=pltpu.CompilerParams(needs_layout_passes=False),
        scratch_types=[
            pltpu.VMEM((128, _CH), jnp.int32),
            pltpu.VMEM((128, _CH), jnp.int32),
            pltpu.VMEM((2, 128), jnp.int32),
            pltpu.VMEM((_CH, 128), f32),
            pltpu.VMEM((_CH, 128), f32),
            pltpu.VMEM((16, 128), f32),
            pltpu.VMEM_SHARED((_ACCR, 128), f32),
            pltpu.SemaphoreType.DMA,
            pltpu.SemaphoreType.DMA,
        ],
    )
    def agg(h0_hbm, h1_hbm, sa_hbm, da_hbm, sb_hbm, db_hbm, cn_hbm,
            o0_hbm, o1_hbm, isv, idv, cv, rows0, rows1, zeros_v, acc,
            sem0, sem1):
        c = lax.axis_index("c")
        s = lax.axis_index("s")
        zv = jnp.zeros((16,), f32)
        for i in range(16):
            for j in range(8):
                zeros_v[i, pl.ds(j * 16, 16)] = zv
        pltpu.sync_copy(cn_hbm.at[pl.ds(2 * s, 2)], cv)
        na0 = jnp.max(cv[0, pl.ds(0, 16)])
        na1 = jnp.max(cv[1, pl.ds(0, 16)])
        nb0 = jnp.max(cv[0, pl.ds(16, 16)])
        nb1 = jnp.max(cv[1, pl.ds(16, 16)])
        rbuf = (rows0, rows1)
        sems = (sem0, sem1)

        def run(h_hbm, o_hbm, s_hbm, d_hbm, p, n0, n1):
            def zero_body(i, _):
                pltpu.sync_copy(zeros_v, acc.at[pl.ds(s * 320 + i * 16, 16)])
                return 0
            lax.fori_loop(0, 20, zero_body, 0)
            plsc.subcore_barrier()

            def start(j, rows, sem):
                pltpu.make_async_copy(h_hbm.at[isv.at[j]], rows, sem).start()

            def wait(j, rows, sem):
                pltpu.make_async_copy(h_hbm.at[isv.at[j]], rows, sem).wait()

            def scat(j, rows):
                pltpu.sync_copy(rows, acc.at[idv.at[j]], add=True)

            for seg in range(2):
                w = 2 * s + seg
                npair = n0 if seg == 0 else n1
                pltpu.sync_copy(s_hbm.at[w], isv)
                pltpu.sync_copy(d_hbm.at[w], idv)

                @pl.when(npair > 0)
                def _():
                    start(0, rows0, sem0)

                    def body(t, _):
                        j0 = 2 * t
                        wait(j0, rows0, sem0)
                        start(j0 + 1, rows1, sem1)
                        scat(j0, rows0)
                        wait(j0 + 1, rows1, sem1)
                        start(j0 + 2, rows0, sem0)
                        scat(j0 + 1, rows1)
                        return 0
                    lax.fori_loop(0, npair - 1, body, 0)
                    last = 2 * npair - 2
                    wait(last, rows0, sem0)
                    start(last + 1, rows1, sem1)
                    scat(last, rows0)
                    wait(last + 1, rows1, sem1)
                    scat(last + 1, rows1)
            plsc.subcore_barrier()
            tail = 304 if p == 0 else 96

            @pl.when(s < _NS - 1)
            def _():
                pltpu.sync_copy(acc.at[pl.ds(s * 320, 320)],
                                o_hbm.at[pl.ds(p * _PASS + s * 320, 320)])

            @pl.when(s == _NS - 1)
            def _():
                pltpu.sync_copy(acc.at[pl.ds(4800, tail)],
                                o_hbm.at[pl.ds(p * _PASS + 4800, tail)])
            plsc.subcore_barrier()

        pl.when(c == 0)(lambda: run(h0_hbm, o0_hbm, sa_hbm, da_hbm, 0, na0, na1))
        pl.when(c == 1)(lambda: run(h1_hbm, o1_hbm, sa_hbm, da_hbm, 0, na0, na1))
        pl.when(c == 0)(lambda: run(h0_hbm, o0_hbm, sb_hbm, db_hbm, 1, nb0, nb1))
        pl.when(c == 1)(lambda: run(h1_hbm, o1_hbm, sb_hbm, db_hbm, 1, nb0, nb1))

    return agg(h0, h1, sa4, da4, sb4, db4, cn)


def _sc_degree(da4, db4, cn, n):
    mesh = plsc.VectorSubcoreMesh(core_axis_name="c", subcore_axis_name="s")
    f32 = jnp.float32

    @functools.partial(
        pl.kernel,
        out_type=(jax.ShapeDtypeStruct((n, 128), f32),
                  jax.ShapeDtypeStruct((n, 128), f32)),
        mesh=mesh,
        compiler_params=pltpu.CompilerParams(needs_layout_passes=False),
        scratch_types=[
            pltpu.VMEM((128, _CH), jnp.int32),
            pltpu.VMEM((128,), jnp.int32),
            pltpu.VMEM((_CH, 128), f32),
            pltpu.VMEM((16, 128), f32),
            pltpu.VMEM_SHARED((_ACCR, 128), f32),
        ],
    )
    def deg(da_hbm, db_hbm, cn_hbm, p0_hbm, p1_hbm, idv, cv, ones_v, zeros_v,
            acc):
        c = lax.axis_index("c")
        s = lax.axis_index("s")
        w = c * _NS + s
        one = jnp.ones((16,), f32)
        zv = jnp.zeros((16,), f32)
        for i in range(_CH):
            for j in range(8):
                ones_v[i, pl.ds(j * 16, 16)] = one
        for i in range(16):
            for j in range(8):
                zeros_v[i, pl.ds(j * 16, 16)] = zv
        pltpu.sync_copy(cn_hbm.at[w], cv)
        na = jnp.max(cv[pl.ds(0, 16)])
        nb = jnp.max(cv[pl.ds(16, 16)])

        def run(o_hbm, d_hbm, nbatch, p):
            def zero_body(i, _):
                pltpu.sync_copy(zeros_v, acc.at[pl.ds(s * 320 + i * 16, 16)])
                return 0
            lax.fori_loop(0, 20, zero_body, 0)
            plsc.subcore_barrier()

            pltpu.sync_copy(d_hbm.at[w], idv)

            def chunk_body(j, _):
                pltpu.sync_copy(ones_v, acc.at[idv.at[j]], add=True)
                return 0
            lax.fori_loop(0, 2 * nbatch, chunk_body, 0)
            plsc.subcore_barrier()
            tail = 304 if p == 0 else 96

            @pl.when(s < _NS - 1)
            def _():
                pltpu.sync_copy(acc.at[pl.ds(s * 320, 320)],
                                o_hbm.at[pl.ds(p * _PASS + s * 320, 320)])

            @pl.when(s == _NS - 1)
            def _():
                pltpu.sync_copy(acc.at[pl.ds(4800, tail)],
                                o_hbm.at[pl.ds(p * _PASS + 4800, tail)])
            plsc.subcore_barrier()

        pl.when(c == 0)(lambda: run(p0_hbm, da_hbm, na, 0))
        pl.when(c == 1)(lambda: run(p1_hbm, da_hbm, na, 0))
        pl.when(c == 0)(lambda: run(p0_hbm, db_hbm, nb, 1))
        pl.when(c == 1)(lambda: run(p1_hbm, db_hbm, nb, 1))

    return deg(da4, db4, cn)


def kernel(x, edge_index, W_in, b_in, W_neigh0, W_self0, b_self0,
           W_neigh1, W_self1, b_self1, W_out, b_out):
    n = x.shape[0]
    e = edge_index.shape[1]
    assert e == _NS * _NCH * _CH and n % _BN == 0 and n <= _ACCR * 2
    src = edge_index[0]
    dst = edge_index[1]
    src2 = src.reshape(2 * _NS, e // (2 * _NS))
    dst2 = dst.reshape(2 * _NS, e // (2 * _NS))

    sa, da, sb, db, cn = _sc_bin(src2, dst2)
    sa4 = sa.reshape(2 * _NS, 128, _CH)
    da4 = da.reshape(2 * _NS, 128, _CH)
    sb4 = sb.reshape(2 * _NS, 128, _CH)
    db4 = db.reshape(2 * _NS, 128, _CH)

    p0, p1 = _sc_degree(da4, db4, cn, n)
    h0, h1 = _tc_in(x, W_in.T, b_in.reshape(1, -1))
    s0, s1 = _sc_agg(h0, h1, sa4, da4, sb4, db4, cn)
    h0, h1 = _tc_mid(h0, h1, s0, s1, p0, p1, W_self0.T, W_neigh0.T,
                     b_self0.reshape(1, -1))
    s0, s1 = _sc_agg(h0, h1, sa4, da4, sb4, db4, cn)
    return _tc_last(h0, h1, s0, s1, p0, p1, W_self1.T, W_neigh1.T,
                    b_self1.reshape(1, -1), W_out.T, b_out.reshape(1, -1))
